# baseline (speedup 1.0000x reference)
import functools

import jax
import jax.numpy as jnp
from jax.experimental import pallas as pl

B, N, T, S = 1024, 148, 64, 32
DIM = 64

BB = 8


def _elu(x):
    return jnp.where(x > 0, x, jnp.exp(x) - 1.0)


def _fused_kernel(phi_ref, y_ref, g_ref, W1_ref, b1_ref, W2_ref, b2_ref,
                  C1_ref, cb1_ref, C2_ref, cb2_ref, C3_ref, cb3_ref, out_ref):
    phi = phi_ref[...]
    yv = y_ref[...]
    g = g_ref[...]

    mn_phi = jnp.min(jnp.where(phi == 0.0, 100.0, phi), axis=2)
    mx_phi = jnp.max(phi, axis=2)
    mn_y = jnp.min(jnp.where(yv == 0.0, 100.0, yv), axis=1)
    mx_y = jnp.max(yv, axis=1)
    mn = jnp.minimum(mn_phi, mn_y)
    mx = jnp.maximum(mx_phi, mx_y)

    eye = (jax.lax.broadcasted_iota(jnp.int32, (N, N), 0) ==
           jax.lax.broadcasted_iota(jnp.int32, (N, N), 1)).astype(jnp.float32)
    A = g + eye[None, :, :]
    deg = jnp.sum(A, axis=2)
    dinv = jax.lax.rsqrt(deg)

    mn_s = mn * dinv
    mx_s = mx * dinv
    p_mn = jnp.sum(A * mn_s[:, None, :], axis=2) * dinv
    p_mx = jnp.sum(A * mx_s[:, None, :], axis=2) * dinv
    W1r0 = W1_ref[0:1, :]
    W1r1 = W1_ref[1:2, :]
    h1 = _elu(p_mn[:, :, None] * W1r0[None] + p_mx[:, :, None] * W1r1[None]
              + b1_ref[...][None])

    hs = h1 * dinv[:, :, None]
    dinv_t = jnp.transpose(dinv)
    W2 = W2_ref[...]
    b2 = b2_ref[...]
    pooled_rows = []
    for j in range(BB):
        a = A[j]
        u = jnp.dot(a, hs[j], preferred_element_type=jnp.float32)
        x2 = dinv_t[:, j:j + 1] * u
        h2 = _elu(jnp.dot(x2, W2, preferred_element_type=jnp.float32) + b2)
        pooled_rows.append(jnp.sum(h2, axis=0, keepdims=True) * (1.0 / N))
    pooled = jnp.concatenate(pooled_rows, axis=0)

    z = _elu(jnp.dot(pooled, C1_ref[...], preferred_element_type=jnp.float32)
             + cb1_ref[...])
    z = _elu(jnp.dot(z, C2_ref[...], preferred_element_type=jnp.float32)
             + cb2_ref[...])
    out_ref[...] = (jnp.dot(z, C3_ref[...], preferred_element_type=jnp.float32)
                    + cb3_ref[...])


@functools.partial(jax.jit, static_argnums=())
def kernel(phi, y, g, W1, b1, W2, b2, C1, cb1, C2, cb2, C3, cb3):
    b1r = b1.reshape(1, -1)
    b2r = b2.reshape(1, -1)
    cb1r = cb1.reshape(1, -1)
    cb2r = cb2.reshape(1, -1)
    cb3r = cb3.reshape(1, -1)

    grid = (B // BB,)
    wspec = lambda shape: pl.BlockSpec(shape, lambda i: (0,) * len(shape))
    out = pl.pallas_call(
        _fused_kernel,
        grid=grid,
        in_specs=[
            pl.BlockSpec((BB, N, T), lambda i: (i, 0, 0)),
            pl.BlockSpec((BB, S, N), lambda i: (i, 0, 0)),
            pl.BlockSpec((BB, N, N), lambda i: (i, 0, 0)),
            wspec(W1.shape),
            wspec(b1r.shape),
            wspec(W2.shape),
            wspec(b2r.shape),
            wspec(C1.shape),
            wspec(cb1r.shape),
            wspec(C2.shape),
            wspec(cb2r.shape),
            wspec(C3.shape),
            wspec(cb3r.shape),
        ],
        out_specs=pl.BlockSpec((BB, 2), lambda i: (i, 0)),
        out_shape=jax.ShapeDtypeStruct((B, 2), jnp.float32),
    )(phi, y, g, W1, b1r, W2, b2r, C1, cb1r, C2, cb2r, C3, cb3r)
    return out


# speedup vs baseline: 1.1098x; 1.1098x over previous
import functools

import jax
import jax.numpy as jnp
from jax.experimental import pallas as pl

B, N, T, S = 1024, 148, 64, 32
DIM = 64

BB = 16


def _elu(x):
    return jnp.where(x > 0, x, jnp.exp(x) - 1.0)


def _fused_kernel(phi_ref, y_ref, g_ref, W1_ref, b1_ref, W2_ref, b2_ref,
                  C1_ref, cb1_ref, C2_ref, cb2_ref, C3_ref, cb3_ref, out_ref):
    phi = phi_ref[...]
    yv = y_ref[...]
    g = g_ref[...]

    mn_phi = jnp.min(jnp.where(phi == 0.0, 100.0, phi), axis=2)
    mx_phi = jnp.max(phi, axis=2)
    mn_y = jnp.min(jnp.where(yv == 0.0, 100.0, yv), axis=1)
    mx_y = jnp.max(yv, axis=1)
    mn = jnp.minimum(mn_phi, mn_y)
    mx = jnp.maximum(mx_phi, mx_y)

    eye = (jax.lax.broadcasted_iota(jnp.int32, (N, N), 0) ==
           jax.lax.broadcasted_iota(jnp.int32, (N, N), 1)).astype(jnp.float32)
    A = g + eye[None, :, :]
    deg = jnp.sum(A, axis=2)
    dinv = jax.lax.rsqrt(deg)

    mn_s = mn * dinv
    mx_s = mx * dinv
    p_mn = jnp.sum(A * mn_s[:, None, :], axis=2) * dinv
    p_mx = jnp.sum(A * mx_s[:, None, :], axis=2) * dinv
    W1r0 = W1_ref[0:1, :]
    W1r1 = W1_ref[1:2, :]
    h1 = _elu(p_mn[:, :, None] * W1r0[None] + p_mx[:, :, None] * W1r1[None]
              + b1_ref[...][None])

    hs = h1 * dinv[:, :, None]
    dinv_t = jnp.transpose(dinv)
    W2 = W2_ref[...]
    b2 = b2_ref[...]
    pooled_rows = []
    for j in range(BB):
        a = A[j]
        u = jnp.dot(a, hs[j], preferred_element_type=jnp.float32)
        x2 = dinv_t[:, j:j + 1] * u
        h2 = _elu(jnp.dot(x2, W2, preferred_element_type=jnp.float32) + b2)
        pooled_rows.append(jnp.sum(h2, axis=0, keepdims=True) * (1.0 / N))
    pooled = jnp.concatenate(pooled_rows, axis=0)

    z = _elu(jnp.dot(pooled, C1_ref[...], preferred_element_type=jnp.float32)
             + cb1_ref[...])
    z = _elu(jnp.dot(z, C2_ref[...], preferred_element_type=jnp.float32)
             + cb2_ref[...])
    out_ref[...] = (jnp.dot(z, C3_ref[...], preferred_element_type=jnp.float32)
                    + cb3_ref[...])


@functools.partial(jax.jit, static_argnums=())
def kernel(phi, y, g, W1, b1, W2, b2, C1, cb1, C2, cb2, C3, cb3):
    b1r = b1.reshape(1, -1)
    b2r = b2.reshape(1, -1)
    cb1r = cb1.reshape(1, -1)
    cb2r = cb2.reshape(1, -1)
    cb3r = cb3.reshape(1, -1)

    grid = (B // BB,)
    wspec = lambda shape: pl.BlockSpec(shape, lambda i: (0,) * len(shape))
    out = pl.pallas_call(
        _fused_kernel,
        grid=grid,
        in_specs=[
            pl.BlockSpec((BB, N, T), lambda i: (i, 0, 0)),
            pl.BlockSpec((BB, S, N), lambda i: (i, 0, 0)),
            pl.BlockSpec((BB, N, N), lambda i: (i, 0, 0)),
            wspec(W1.shape),
            wspec(b1r.shape),
            wspec(W2.shape),
            wspec(b2r.shape),
            wspec(C1.shape),
            wspec(cb1r.shape),
            wspec(C2.shape),
            wspec(cb2r.shape),
            wspec(C3.shape),
            wspec(cb3r.shape),
        ],
        out_specs=pl.BlockSpec((BB, 2), lambda i: (i, 0)),
        out_shape=jax.ShapeDtypeStruct((B, 2), jnp.float32),
    )(phi, y, g, W1, b1r, W2, b2r, C1, cb1r, C2, cb2r, C3, cb3r)
    return out


# speedup vs baseline: 1.1860x; 1.0686x over previous
import functools

import jax
import jax.numpy as jnp
from jax.experimental import pallas as pl

B, N, T, S = 1024, 148, 64, 32
DIM = 64

BB = 32


def _elu(x):
    return jnp.where(x > 0, x, jnp.exp(x) - 1.0)


def _fused_kernel(phi_ref, y_ref, g_ref, W1_ref, b1_ref, W2_ref, b2_ref,
                  C1_ref, cb1_ref, C2_ref, cb2_ref, C3_ref, cb3_ref, out_ref):
    phi = phi_ref[...]
    yv = y_ref[...]
    g = g_ref[...]

    mn_phi = jnp.min(jnp.where(phi == 0.0, 100.0, phi), axis=2)
    mx_phi = jnp.max(phi, axis=2)
    mn_y = jnp.min(jnp.where(yv == 0.0, 100.0, yv), axis=1)
    mx_y = jnp.max(yv, axis=1)
    mn = jnp.minimum(mn_phi, mn_y)
    mx = jnp.maximum(mx_phi, mx_y)

    eye = (jax.lax.broadcasted_iota(jnp.int32, (N, N), 0) ==
           jax.lax.broadcasted_iota(jnp.int32, (N, N), 1)).astype(jnp.float32)
    A = g + eye[None, :, :]
    deg = jnp.sum(A, axis=2)
    dinv = jax.lax.rsqrt(deg)

    mn_s = mn * dinv
    mx_s = mx * dinv
    p_mn = jnp.sum(A * mn_s[:, None, :], axis=2) * dinv
    p_mx = jnp.sum(A * mx_s[:, None, :], axis=2) * dinv
    W1r0 = W1_ref[0:1, :]
    W1r1 = W1_ref[1:2, :]
    h1 = _elu(p_mn[:, :, None] * W1r0[None] + p_mx[:, :, None] * W1r1[None]
              + b1_ref[...][None])

    hs = h1 * dinv[:, :, None]
    dinv_t = jnp.transpose(dinv)
    W2 = W2_ref[...]
    b2 = b2_ref[...]
    pooled_rows = []
    for j in range(BB):
        a = A[j]
        u = jnp.dot(a, hs[j], preferred_element_type=jnp.float32)
        x2 = dinv_t[:, j:j + 1] * u
        h2 = _elu(jnp.dot(x2, W2, preferred_element_type=jnp.float32) + b2)
        pooled_rows.append(jnp.sum(h2, axis=0, keepdims=True) * (1.0 / N))
    pooled = jnp.concatenate(pooled_rows, axis=0)

    z = _elu(jnp.dot(pooled, C1_ref[...], preferred_element_type=jnp.float32)
             + cb1_ref[...])
    z = _elu(jnp.dot(z, C2_ref[...], preferred_element_type=jnp.float32)
             + cb2_ref[...])
    out_ref[...] = (jnp.dot(z, C3_ref[...], preferred_element_type=jnp.float32)
                    + cb3_ref[...])


@functools.partial(jax.jit, static_argnums=())
def kernel(phi, y, g, W1, b1, W2, b2, C1, cb1, C2, cb2, C3, cb3):
    b1r = b1.reshape(1, -1)
    b2r = b2.reshape(1, -1)
    cb1r = cb1.reshape(1, -1)
    cb2r = cb2.reshape(1, -1)
    cb3r = cb3.reshape(1, -1)

    grid = (B // BB,)
    wspec = lambda shape: pl.BlockSpec(shape, lambda i: (0,) * len(shape))
    out = pl.pallas_call(
        _fused_kernel,
        grid=grid,
        in_specs=[
            pl.BlockSpec((BB, N, T), lambda i: (i, 0, 0)),
            pl.BlockSpec((BB, S, N), lambda i: (i, 0, 0)),
            pl.BlockSpec((BB, N, N), lambda i: (i, 0, 0)),
            wspec(W1.shape),
            wspec(b1r.shape),
            wspec(W2.shape),
            wspec(b2r.shape),
            wspec(C1.shape),
            wspec(cb1r.shape),
            wspec(C2.shape),
            wspec(cb2r.shape),
            wspec(C3.shape),
            wspec(cb3r.shape),
        ],
        out_specs=pl.BlockSpec((BB, 2), lambda i: (i, 0)),
        out_shape=jax.ShapeDtypeStruct((B, 2), jnp.float32),
    )(phi, y, g, W1, b1r, W2, b2r, C1, cb1r, C2, cb2r, C3, cb3r)
    return out
